# baseline (device time: 148652 ns/iter reference)
import jax
import jax.numpy as jnp
from jax import lax
from jax.experimental import pallas as pl
from jax.experimental.pallas import tpu as pltpu

N_DEV = 32
N_GRP = 16
B = 2
S = 128
HD = 256
D_OUT = 512


def kernel(x, Wq, K_ext, V_ext, Wo):
    K2 = K_ext.reshape(B, S, HD)
    V2 = V_ext.reshape(B, S, HD)

    def body(x_ref, wq_ref, k_ref, v_ref, wo_ref, out_ref,
             kv_all, q_scr, ctx_scr,
             send_r, recv_r, send_l, recv_l):
        my = lax.axis_index("i")
        right = lax.rem(my + 2, N_DEV)
        left = lax.rem(my + N_DEV - 2, N_DEV)

        barrier_sem = pltpu.get_barrier_semaphore()
        for nbr in (left, right):
            pl.semaphore_signal(
                barrier_sem, inc=1,
                device_id=(nbr,), device_id_type=pl.DeviceIdType.MESH,
            )
        pl.semaphore_wait(barrier_sem, 2)

        kv_all[0, 0] = k_ref[...]
        kv_all[0, 1] = v_ref[...]

        xm = x_ref[...].reshape(B * S, D_OUT)
        q = jnp.dot(xm, wq_ref[...], preferred_element_type=jnp.float32)
        q_scr[...] = (q * 0.125).reshape(B, S, HD)

        for h in range(8):
            rdma_r = pltpu.make_async_remote_copy(
                src_ref=kv_all.at[h],
                dst_ref=kv_all.at[h + 1],
                send_sem=send_r.at[h],
                recv_sem=recv_r.at[h],
                device_id=(right,),
                device_id_type=pl.DeviceIdType.MESH,
            )
            rdma_r.start()
            rdma_l = None
            if h < 7:
                rdma_l = pltpu.make_async_remote_copy(
                    src_ref=kv_all.at[0 if h == 0 else 8 + h],
                    dst_ref=kv_all.at[9 + h],
                    send_sem=send_l.at[h],
                    recv_sem=recv_l.at[h],
                    device_id=(left,),
                    device_id_type=pl.DeviceIdType.MESH,
                )
                rdma_l.start()
            rdma_r.wait()
            if rdma_l is not None:
                rdma_l.wait()

        for b in range(B):
            for hh in range(4):
                for blk in range(2):
                    rows = pl.ds(blk * 64, 64)
                    cols = pl.ds(hh * 64, 64)
                    qt = q_scr[b, rows, cols]
                    kt = kv_all[:, 0, b, rows, cols]
                    vt = kv_all[:, 1, b, rows, cols]
                    kt = kt.reshape(N_GRP * 64, 64)
                    vt = vt.reshape(N_GRP * 64, 64)
                    s = lax.dot_general(
                        qt, kt, (((1,), (1,)), ((), ())),
                        preferred_element_type=jnp.float32,
                    )
                    m = jnp.max(s, axis=-1, keepdims=True)
                    w = jnp.exp(s - m)
                    w = w / jnp.sum(w, axis=-1, keepdims=True)
                    ctx = jnp.dot(w, vt, preferred_element_type=jnp.float32)
                    ctx_scr[b, rows, cols] = ctx

        out = jnp.dot(ctx_scr[...].reshape(B * S, HD), wo_ref[...],
                      preferred_element_type=jnp.float32)
        out_ref[...] = out.reshape(B, S, D_OUT)

    return pl.pallas_call(
        body,
        out_shape=jax.ShapeDtypeStruct((B, S, D_OUT), jnp.float32),
        in_specs=[pl.BlockSpec(memory_space=pltpu.VMEM)] * 5,
        out_specs=pl.BlockSpec(memory_space=pltpu.VMEM),
        scratch_shapes=[
            pltpu.VMEM((N_GRP, 2, B, S, HD), jnp.float32),
            pltpu.VMEM((B, S, HD), jnp.float32),
            pltpu.VMEM((B, S, HD), jnp.float32),
            pltpu.SemaphoreType.DMA((8,)),
            pltpu.SemaphoreType.DMA((8,)),
            pltpu.SemaphoreType.DMA((7,)),
            pltpu.SemaphoreType.DMA((7,)),
        ],
        compiler_params=pltpu.CompilerParams(collective_id=0),
    )(x, Wq, K2, V2, Wo)


# device time: 107281 ns/iter; 1.3856x vs baseline; 1.3856x over previous
import jax
import jax.numpy as jnp
from jax import lax
from jax.experimental import pallas as pl
from jax.experimental.pallas import tpu as pltpu

N_DEV = 32
N_GRP = 16
B = 2
S = 128
HD = 256
D_OUT = 512


def kernel(x, Wq, K_ext, V_ext, Wo):
    K2 = K_ext.reshape(B, S, HD)
    V2 = V_ext.reshape(B, S, HD)

    def body(x_ref, wq_ref, k_ref, v_ref, wo_ref, out_ref,
             kv_all, q_scr, ctx_scr,
             send_r, recv_r, send_l, recv_l):
        my = lax.axis_index("i")
        right = lax.rem(my + 2, N_DEV)
        left = lax.rem(my + N_DEV - 2, N_DEV)

        barrier_sem = pltpu.get_barrier_semaphore()
        for nbr in (left, right):
            pl.semaphore_signal(
                barrier_sem, inc=1,
                device_id=(nbr,), device_id_type=pl.DeviceIdType.MESH,
            )
        pl.semaphore_wait(barrier_sem, 2)

        kv_all[0, 0] = k_ref[...]
        kv_all[0, 1] = v_ref[...]

        xm = x_ref[...].reshape(B * S, D_OUT)
        q = jnp.dot(xm, wq_ref[...], preferred_element_type=jnp.float32)
        q_scr[...] = (q * 0.125).reshape(B, S, HD)

        k_rdmas = []
        v_rdmas = []
        for h in range(N_GRP - 1):
            if h > 0:
                k_rdmas[h - 1].wait_recv()
            rk = pltpu.make_async_remote_copy(
                src_ref=kv_all.at[h, 0],
                dst_ref=kv_all.at[h + 1, 0],
                send_sem=send_r.at[h],
                recv_sem=recv_r.at[h],
                device_id=(right,),
                device_id_type=pl.DeviceIdType.MESH,
            )
            rk.start()
            k_rdmas.append(rk)
            if h > 0:
                v_rdmas[h - 1].wait_recv()
            rv = pltpu.make_async_remote_copy(
                src_ref=kv_all.at[h, 1],
                dst_ref=kv_all.at[h + 1, 1],
                send_sem=send_l.at[h],
                recv_sem=recv_l.at[h],
                device_id=(right,),
                device_id_type=pl.DeviceIdType.MESH,
            )
            rv.start()
            v_rdmas.append(rv)
        k_rdmas[-1].wait_recv()
        v_rdmas[-1].wait_recv()
        for h in range(N_GRP - 1):
            k_rdmas[h].wait_send()
            v_rdmas[h].wait_send()

        for b in range(B):
            for hh in range(4):
                for blk in range(2):
                    rows = pl.ds(blk * 64, 64)
                    cols = pl.ds(hh * 64, 64)
                    qt = q_scr[b, rows, cols]
                    kt = kv_all[:, 0, b, rows, cols]
                    vt = kv_all[:, 1, b, rows, cols]
                    kt = kt.reshape(N_GRP * 64, 64)
                    vt = vt.reshape(N_GRP * 64, 64)
                    s = lax.dot_general(
                        qt, kt, (((1,), (1,)), ((), ())),
                        preferred_element_type=jnp.float32,
                    )
                    m = jnp.max(s, axis=-1, keepdims=True)
                    w = jnp.exp(s - m)
                    w = w / jnp.sum(w, axis=-1, keepdims=True)
                    ctx = jnp.dot(w, vt, preferred_element_type=jnp.float32)
                    ctx_scr[b, rows, cols] = ctx

        out = jnp.dot(ctx_scr[...].reshape(B * S, HD), wo_ref[...],
                      preferred_element_type=jnp.float32)
        out_ref[...] = out.reshape(B, S, D_OUT)

    return pl.pallas_call(
        body,
        out_shape=jax.ShapeDtypeStruct((B, S, D_OUT), jnp.float32),
        in_specs=[pl.BlockSpec(memory_space=pltpu.VMEM)] * 5,
        out_specs=pl.BlockSpec(memory_space=pltpu.VMEM),
        scratch_shapes=[
            pltpu.VMEM((N_GRP, 2, B, S, HD), jnp.float32),
            pltpu.VMEM((B, S, HD), jnp.float32),
            pltpu.VMEM((B, S, HD), jnp.float32),
            pltpu.SemaphoreType.DMA((N_GRP - 1,)),
            pltpu.SemaphoreType.DMA((N_GRP - 1,)),
            pltpu.SemaphoreType.DMA((N_GRP - 1,)),
            pltpu.SemaphoreType.DMA((N_GRP - 1,)),
        ],
        compiler_params=pltpu.CompilerParams(collective_id=0),
    )(x, Wq, K2, V2, Wo)
